# Initial kernel scaffold; baseline (speedup 1.0000x reference)
import jax
import jax.numpy as jnp
from jax.experimental import pallas as pl


def kernel(cut_embedding, local_clusterxvariant_indptr, n_variants, n_clusters, cluster_cut_lib):
    raise NotImplementedError("write your pallas kernel here")


# speedup vs baseline: 76.8128x; 76.8128x over previous
import functools

import jax
import jax.numpy as jnp
from jax import lax
from jax.experimental import pallas as pl
from jax.experimental.pallas import tpu as pltpu
from jax.experimental.pallas import tpu_sc as plsc

_PB = 1280


def _prefix_body(x_ref, out_ref, carry_ref):
    @pl.when(pl.program_id(0) == 0)
    def _():
        carry_ref[...] = jnp.zeros_like(carry_ref)

    x = x_ref[...] - 0.5
    c = x
    k = 1
    nrow, ncol = x.shape
    while k < nrow:
        c = c + jnp.concatenate(
            [jnp.zeros((k, ncol), c.dtype), c[: nrow - k, :]], axis=0
        )
        k *= 2
    out_ref[...] = carry_ref[0:1, :] + (c - x)
    carry_ref[0:1, :] = carry_ref[0:1, :] + c[nrow - 1 : nrow, :]


def _prefix_sum_centered(x):
    n, d = x.shape
    nb = n // _PB
    return pl.pallas_call(
        _prefix_body,
        grid=(nb,),
        in_specs=[pl.BlockSpec((_PB, d), lambda i: (i, 0))],
        out_specs=pl.BlockSpec((_PB, d), lambda i: (i, 0)),
        out_shape=jax.ShapeDtypeStruct((n, d), jnp.float32),
        scratch_shapes=[pltpu.VMEM((8, d), jnp.float32)],
        compiler_params=pltpu.CompilerParams(dimension_semantics=("arbitrary",)),
    )(x)


_SEG_CHUNK = 400


def _make_sc_gather_diff(n_rows, d, n_seg):
    info = plsc.get_sparse_core_info()
    n_cores, n_sub = info.num_cores, info.num_subcores
    nw = n_cores * n_sub
    c_sz = _SEG_CHUNK
    gp = c_sz + 8
    nchunk = n_seg // c_sz
    mesh = plsc.VectorSubcoreMesh(core_axis_name="c", subcore_axis_name="s")

    @functools.partial(
        pl.kernel,
        mesh=mesh,
        out_type=jax.ShapeDtypeStruct((n_seg, d), jnp.float32),
        scratch_types=[
            pltpu.VMEM((gp,), jnp.int32),
            pltpu.VMEM((gp, d), jnp.float32),
            pltpu.VMEM((c_sz, d), jnp.float32),
            pltpu.SemaphoreType.DMA,
        ],
    )
    def k(p_hbm, idx_hbm, out_hbm, idx_v, g_v, d_v, sem):
        wid = lax.axis_index("s") * n_cores + lax.axis_index("c")
        rem = nchunk % nw
        nloc = jnp.where(wid < rem, nchunk // nw + 1, nchunk // nw)

        def chunk_body(j, carry):
            chunk = wid + j * nw
            base = chunk * c_sz
            pltpu.sync_copy(idx_hbm.at[pl.ds(base, gp)], idx_v)
            pltpu.async_copy(p_hbm.at[idx_v], g_v, sem).wait()

            def row_body(r, c2):
                for g8 in range(d // 16):
                    off = g8 * 16
                    d_v[r, pl.ds(off, 16)] = (
                        g_v[r + 1, pl.ds(off, 16)] - g_v[r, pl.ds(off, 16)]
                    )
                return c2

            lax.fori_loop(0, c_sz, row_body, 0)
            pltpu.sync_copy(d_v, out_hbm.at[pl.ds(base, c_sz)])
            return carry

        lax.fori_loop(0, nloc, chunk_body, 0)

    return k


_VT = 200


def _norm_body(s_ref, len_ref, lib_ref, out_ref):
    s = s_ref[...] + 0.5 * len_ref[...]
    ve = jnp.log1p(s / lib_ref[...][:, None, :]) - 2.0
    n_c = ve.shape[0]
    mu = jnp.mean(ve, axis=0, keepdims=True)
    sd = jnp.sqrt(jnp.sum((ve - mu) ** 2, axis=0, keepdims=True) / (n_c - 1))
    rel = (ve - mu) / (sd + 1e-5)
    d = ve.shape[-1]
    out_ref[..., 0:d] = ve
    out_ref[..., d : 2 * d] = rel


def _normalize(seg_sums3, lens3, lib2d):
    n_c, n_v, d = seg_sums3.shape
    nt = n_v // _VT
    return pl.pallas_call(
        _norm_body,
        grid=(nt,),
        in_specs=[
            pl.BlockSpec((n_c, _VT, d), lambda t: (0, t, 0)),
            pl.BlockSpec((n_c, _VT, 1), lambda t: (0, t, 0)),
            pl.BlockSpec((n_c, d), lambda t: (0, 0)),
        ],
        out_specs=pl.BlockSpec((n_c, _VT, 2 * d), lambda t: (0, t, 0)),
        out_shape=jax.ShapeDtypeStruct((n_c, n_v, 2 * d), jnp.float32),
        compiler_params=pltpu.CompilerParams(dimension_semantics=("parallel",)),
    )(seg_sums3, lens3, lib2d)


def kernel(cut_embedding, local_clusterxvariant_indptr, n_variants, n_clusters, cluster_cut_lib):
    n_rows, d = cut_embedding.shape
    n_c = cluster_cut_lib.shape[0]
    n_seg = local_clusterxvariant_indptr.shape[0] - 1
    n_v = n_seg // n_c

    idx = local_clusterxvariant_indptr.astype(jnp.int32)
    pad = 8 - (idx.shape[0] % 8) if idx.shape[0] % 8 else 0
    idx_pad = jnp.concatenate([idx, jnp.broadcast_to(idx[-1:], (pad,))])
    lens3 = (idx[1:] - idx[:-1]).astype(jnp.float32).reshape(n_c, n_v, 1)
    lib2d = jnp.broadcast_to(
        cluster_cut_lib.astype(jnp.float32)[:, None], (n_c, d)
    )

    p_centered = _prefix_sum_centered(cut_embedding.astype(jnp.float32))
    seg_sums = _make_sc_gather_diff(n_rows, d, n_seg)(p_centered, idx_pad)
    return _normalize(seg_sums.reshape(n_c, n_v, d), lens3, lib2d)
